# baseline (speedup 1.0000x reference)
import functools

import jax
import jax.numpy as jnp
from jax import lax
from jax.experimental import pallas as pl
from jax.experimental.pallas import tpu as pltpu
from jax.experimental.pallas import tpu_sc as plsc

_BM = 512
_CH = 64


def _sc_gather3(src_emb, dst_emb, h_idx, t_idx, n_idx):
    M = h_idx.shape[0]
    D = src_emb.shape[1]
    info = plsc.get_sparse_core_info()
    NC, NS = info.num_cores, info.num_subcores
    NW = NC * NS
    BPW = M // NW
    NCH = BPW // _CH
    NPAIR = NCH // 2
    mesh = plsc.VectorSubcoreMesh(core_axis_name="c", subcore_axis_name="s")

    @functools.partial(
        pl.kernel,
        mesh=mesh,
        out_type=[jax.ShapeDtypeStruct((M, D), jnp.float32)] * 3,
        scratch_types=[
            pltpu.VMEM((BPW,), jnp.int32),
            pltpu.VMEM((_CH, D), jnp.float32),
            pltpu.VMEM((_CH, D), jnp.float32),
            pltpu.SemaphoreType.DMA,
            pltpu.SemaphoreType.DMA,
        ],
    )
    def k(src_hbm, dst_hbm, hi_hbm, ti_hbm, ni_hbm,
          out_h, out_t, out_n, idx_v, buf0, buf1, sem0, sem1):
        wid = lax.axis_index("s") * NC + lax.axis_index("c")
        base = wid * BPW

        def gather_one(table, idx_hbm, out):
            pltpu.sync_copy(idx_hbm.at[pl.ds(base, BPW)], idx_v)

            def fire(c, buf, sem):
                pltpu.async_copy(table.at[idx_v.at[pl.ds(c * _CH, _CH)]],
                                 buf, sem)

            def drain(buf, sem):
                pltpu.make_async_copy(table.at[idx_v.at[pl.ds(0, _CH)]],
                                      buf, sem).wait()

            fire(0, buf0, sem0)

            def pair_body(p, carry):
                c0 = p * 2
                c1 = c0 + 1
                fire(c1, buf1, sem1)
                drain(buf0, sem0)
                pltpu.sync_copy(buf0, out.at[pl.ds(base + c0 * _CH, _CH)])

                @pl.when(p + 1 < NPAIR)
                def _():
                    fire(c0 + 2, buf0, sem0)

                drain(buf1, sem1)
                pltpu.sync_copy(buf1, out.at[pl.ds(base + c1 * _CH, _CH)])
                return carry

            lax.fori_loop(0, NPAIR, pair_body, 0)

        gather_one(src_hbm, hi_hbm, out_h)
        gather_one(dst_hbm, ti_hbm, out_t)
        gather_one(dst_hbm, ni_hbm, out_n)

    return k(src_emb, dst_emb, h_idx, t_idx, n_idx)


def _tc_body(h_ref, t_ref, n_ref, et_ref, wab_ref, wbb_ref, w1b_ref,
             b1_ref, br1_ref, w2t_ref, b2_ref, wr2_ref, br2_ref, out_ref,
             *, D, R, M):
    pid = pl.program_id(0)
    h = h_ref[...].astype(jnp.bfloat16)
    t = t_ref[...].astype(jnp.bfloat16)
    n = n_ref[...].astype(jnp.bfloat16)
    wab = wab_ref[...].astype(jnp.bfloat16)
    wbb = wbb_ref[...].astype(jnp.bfloat16)
    w1b = w1b_ref[...].astype(jnp.bfloat16)
    th = jnp.dot(h, wab, preferred_element_type=jnp.float32)
    tt = jnp.dot(t, wbb, preferred_element_type=jnp.float32)
    tn = jnp.dot(n, w1b, preferred_element_type=jnp.float32)
    b1 = b1_ref[...]
    br1 = br1_ref[...]
    t1 = th[:, :D]
    r1h = th[:, D:]
    t2 = tt[:, :D]
    r1t = tt[:, D:]
    h_pos = jnp.maximum(t1 + t2 + b1, 0.0)
    h_neg = jnp.maximum(t1 + tn + b1, 0.0)
    r1 = jnp.maximum(r1h + r1t + br1, 0.0)

    w2t = w2t_ref[...]
    b2 = b2_ref[0, 0]
    lp = jnp.sum(h_pos * w2t, axis=1, keepdims=True) + b2
    ln = jnp.sum(h_neg * w2t, axis=1, keepdims=True) + b2

    def softplus(x):
        return jnp.maximum(x, 0.0) + jnp.log(1.0 + jnp.exp(-jnp.abs(x)))

    bce_part = jnp.sum(softplus(lp) - lp) + jnp.sum(softplus(ln))

    rel = jnp.dot(r1, wr2_ref[...], preferred_element_type=jnp.float32)
    rel = rel + br2_ref[...]
    mx = jnp.max(rel, axis=1, keepdims=True)
    lse = mx + jnp.log(jnp.sum(jnp.exp(rel - mx), axis=1, keepdims=True))
    et = et_ref[0]
    onehot = (lax.broadcasted_iota(jnp.int32, rel.shape, 1) == et)
    picked = jnp.sum(jnp.where(onehot, rel, 0.0), axis=1, keepdims=True)
    ce_part = jnp.sum(lse - picked)

    contrib = (bce_part / (2.0 * M) + ce_part / M) * 0.5

    @pl.when(pid == 0)
    def _():
        out_ref[0, 0] = 0.0

    out_ref[0, 0] += contrib


def _tc_loss(heads, tails, negs, et3, wab, wbb, w1b, b1r, br1r, w2t, b2s,
             wr2, br2r):
    M, D = heads.shape
    R = wr2.shape[1]
    nblk = M // _BM
    body = functools.partial(_tc_body, D=D, R=R, M=M)
    out = pl.pallas_call(
        body,
        grid=(nblk,),
        in_specs=[
            pl.BlockSpec((_BM, D), lambda i: (i, 0)),
            pl.BlockSpec((_BM, D), lambda i: (i, 0)),
            pl.BlockSpec((_BM, D), lambda i: (i, 0)),
            pl.BlockSpec((1, _BM, 1), lambda i: (i, 0, 0)),
            pl.BlockSpec((D, 2 * D), lambda i: (0, 0)),
            pl.BlockSpec((D, 2 * D), lambda i: (0, 0)),
            pl.BlockSpec((D, D), lambda i: (0, 0)),
            pl.BlockSpec((1, D), lambda i: (0, 0)),
            pl.BlockSpec((1, D), lambda i: (0, 0)),
            pl.BlockSpec((1, D), lambda i: (0, 0)),
            pl.BlockSpec(memory_space=pltpu.SMEM),
            pl.BlockSpec((D, R), lambda i: (0, 0)),
            pl.BlockSpec((1, R), lambda i: (0, 0)),
        ],
        out_specs=pl.BlockSpec(memory_space=pltpu.SMEM),
        out_shape=jax.ShapeDtypeStruct((1, 1), jnp.float32),
    )(heads, tails, negs, et3, wab, wbb, w1b, b1r, br1r, w2t, b2s, wr2, br2r)
    return out


def kernel(src_emb, dst_emb, edge_index, edge_type_idx, neg_tail_idx,
           W1, b1, W2, b2, Wr1, br1, Wr2, br2):
    M = edge_index.shape[1]
    D = src_emb.shape[1]
    R = Wr2.shape[1]

    h_idx = edge_index[0].astype(jnp.int32)
    t_idx = edge_index[1].astype(jnp.int32)
    n_idx = neg_tail_idx.astype(jnp.int32)

    heads, tails, negs = _sc_gather3(src_emb, dst_emb, h_idx, t_idx, n_idx)

    wab = jnp.concatenate([W1[:D], Wr1[:D]], axis=1)
    wbb = jnp.concatenate([W1[D:], Wr1[D:]], axis=1)
    w1b = W1[D:]
    b1r = b1.reshape(1, D)
    br1r = br1.reshape(1, D)
    w2t = W2.reshape(1, D)
    b2s = b2.reshape(1, 1)
    br2r = br2.reshape(1, R)
    et3 = edge_type_idx.astype(jnp.int32).reshape(M // _BM, _BM, 1)

    out = _tc_loss(heads, tails, negs, et3, wab, wbb, w1b, b1r, br1r,
                   w2t, b2s, wr2=Wr2, br2r=br2r)
    return out.reshape(())


# speedup vs baseline: 1.0941x; 1.0941x over previous
import functools

import jax
import jax.numpy as jnp
from jax import lax
from jax.experimental import pallas as pl
from jax.experimental.pallas import tpu as pltpu
from jax.experimental.pallas import tpu_sc as plsc

_BM = 512
_CH = 64


def _sc_gather3(src_emb, dst_emb, h_idx, t_idx, n_idx):
    M = h_idx.shape[0]
    D = src_emb.shape[1]
    info = plsc.get_sparse_core_info()
    NC, NS = info.num_cores, info.num_subcores
    NW = NC * NS
    BPW = M // NW
    NCH = BPW // _CH
    NPAIR = NCH // 2
    mesh = plsc.VectorSubcoreMesh(core_axis_name="c", subcore_axis_name="s")

    @functools.partial(
        pl.kernel,
        mesh=mesh,
        out_type=[jax.ShapeDtypeStruct((M, D), jnp.float32)] * 3,
        scratch_types=[
            pltpu.VMEM((BPW,), jnp.int32),
            pltpu.VMEM((_CH, D), jnp.float32),
            pltpu.VMEM((_CH, D), jnp.float32),
            pltpu.SemaphoreType.DMA,
            pltpu.SemaphoreType.DMA,
        ],
    )
    def k(src_hbm, dst_hbm, hi_hbm, ti_hbm, ni_hbm,
          out_h, out_t, out_n, idx_v, buf0, buf1, sem0, sem1):
        wid = lax.axis_index("s") * NC + lax.axis_index("c")
        base = wid * BPW

        def gather_one(table, idx_hbm, out):
            pltpu.sync_copy(idx_hbm.at[pl.ds(base, BPW)], idx_v)

            def fire(c, buf, sem):
                pltpu.async_copy(table.at[idx_v.at[pl.ds(c * _CH, _CH)]],
                                 buf, sem)

            def drain(buf, sem):
                pltpu.make_async_copy(table.at[idx_v.at[pl.ds(0, _CH)]],
                                      buf, sem).wait()

            fire(0, buf0, sem0)

            def pair_body(p, carry):
                c0 = p * 2
                c1 = c0 + 1
                fire(c1, buf1, sem1)
                drain(buf0, sem0)
                pltpu.sync_copy(buf0, out.at[pl.ds(base + c0 * _CH, _CH)])

                @pl.when(p + 1 < NPAIR)
                def _():
                    fire(c0 + 2, buf0, sem0)

                drain(buf1, sem1)
                pltpu.sync_copy(buf1, out.at[pl.ds(base + c1 * _CH, _CH)])
                return carry

            lax.fori_loop(0, NPAIR, pair_body, 0)

        gather_one(src_hbm, hi_hbm, out_h)
        gather_one(dst_hbm, ti_hbm, out_t)
        gather_one(dst_hbm, ni_hbm, out_n)

    return k(src_emb, dst_emb, h_idx, t_idx, n_idx)


def _tc_body(h_ref, t_ref, n_ref, et_ref, wab_ref, wbb_ref, w1b_ref,
             b1_ref, br1_ref, w2t_ref, b2_ref, wr2_ref, br2_ref, out_ref,
             *, D, R, M):
    pid = pl.program_id(0)
    h = h_ref[...].astype(jnp.bfloat16)
    t = t_ref[...].astype(jnp.bfloat16)
    n = n_ref[...].astype(jnp.bfloat16)
    wab = wab_ref[...].astype(jnp.bfloat16)
    wbb = wbb_ref[...].astype(jnp.bfloat16)
    w1b = w1b_ref[...].astype(jnp.bfloat16)
    th = jnp.dot(h, wab, preferred_element_type=jnp.float32)
    tt = jnp.dot(t, wbb, preferred_element_type=jnp.float32)
    tn = jnp.dot(n, w1b, preferred_element_type=jnp.float32)
    b1 = b1_ref[...]
    br1 = br1_ref[...]
    t1 = th[:, :D]
    r1h = th[:, D:]
    t2 = tt[:, :D]
    r1t = tt[:, D:]
    h_pos = jnp.maximum(t1 + t2 + b1, 0.0)
    h_neg = jnp.maximum(t1 + tn + b1, 0.0)
    r1 = jnp.maximum(r1h + r1t + br1, 0.0)

    w2t = w2t_ref[...]
    b2 = b2_ref[0, 0]
    lp = jnp.sum(h_pos * w2t, axis=1, keepdims=True) + b2
    ln = jnp.sum(h_neg * w2t, axis=1, keepdims=True) + b2

    def softplus(x):
        return jnp.maximum(x, 0.0) + jnp.log(1.0 + jnp.exp(-jnp.abs(x)))

    bce_part = jnp.sum(softplus(lp) - lp) + jnp.sum(softplus(ln))

    rel = jnp.dot(r1, wr2_ref[...], preferred_element_type=jnp.float32)
    rel = rel + br2_ref[...]
    mx = jnp.max(rel, axis=1, keepdims=True)
    lse = mx + jnp.log(jnp.sum(jnp.exp(rel - mx), axis=1, keepdims=True))
    et = et_ref[0]
    onehot = (lax.broadcasted_iota(jnp.int32, rel.shape, 1) == et)
    picked = jnp.sum(jnp.where(onehot, rel, 0.0), axis=1, keepdims=True)
    ce_part = jnp.sum(lse - picked)

    contrib = (bce_part / (2.0 * M) + ce_part / M) * 0.5

    @pl.when(pid == 0)
    def _():
        out_ref[0, 0] = 0.0

    out_ref[0, 0] += contrib


def _tc_loss(heads, tails, negs, et3, wab, wbb, w1b, b1r, br1r, w2t, b2s,
             wr2, br2r, m_total):
    M, D = heads.shape
    R = wr2.shape[1]
    nblk = M // _BM
    body = functools.partial(_tc_body, D=D, R=R, M=m_total)
    out = pl.pallas_call(
        body,
        grid=(nblk,),
        in_specs=[
            pl.BlockSpec((_BM, D), lambda i: (i, 0)),
            pl.BlockSpec((_BM, D), lambda i: (i, 0)),
            pl.BlockSpec((_BM, D), lambda i: (i, 0)),
            pl.BlockSpec((1, _BM, 1), lambda i: (i, 0, 0)),
            pl.BlockSpec((D, 2 * D), lambda i: (0, 0)),
            pl.BlockSpec((D, 2 * D), lambda i: (0, 0)),
            pl.BlockSpec((D, D), lambda i: (0, 0)),
            pl.BlockSpec((1, D), lambda i: (0, 0)),
            pl.BlockSpec((1, D), lambda i: (0, 0)),
            pl.BlockSpec((1, D), lambda i: (0, 0)),
            pl.BlockSpec(memory_space=pltpu.SMEM),
            pl.BlockSpec((D, R), lambda i: (0, 0)),
            pl.BlockSpec((1, R), lambda i: (0, 0)),
        ],
        out_specs=pl.BlockSpec(memory_space=pltpu.SMEM),
        out_shape=jax.ShapeDtypeStruct((1, 1), jnp.float32),
    )(heads, tails, negs, et3, wab, wbb, w1b, b1r, br1r, w2t, b2s, wr2, br2r)
    return out


def kernel(src_emb, dst_emb, edge_index, edge_type_idx, neg_tail_idx,
           W1, b1, W2, b2, Wr1, br1, Wr2, br2):
    M = edge_index.shape[1]
    D = src_emb.shape[1]
    R = Wr2.shape[1]

    h_idx = edge_index[0].astype(jnp.int32)
    t_idx = edge_index[1].astype(jnp.int32)
    n_idx = neg_tail_idx.astype(jnp.int32)
    et = edge_type_idx.astype(jnp.int32)

    wab = jnp.concatenate([W1[:D], Wr1[:D]], axis=1)
    wbb = jnp.concatenate([W1[D:], Wr1[D:]], axis=1)
    w1b = W1[D:]
    b1r = b1.reshape(1, D)
    br1r = br1.reshape(1, D)
    w2t = W2.reshape(1, D)
    b2s = b2.reshape(1, 1)
    br2r = br2.reshape(1, R)

    S = 4
    MS = M // S
    out = jnp.zeros((1, 1), jnp.float32)
    for s in range(S):
        heads, tails, negs = _sc_gather3(
            src_emb, dst_emb,
            lax.dynamic_slice(h_idx, (s * MS,), (MS,)),
            lax.dynamic_slice(t_idx, (s * MS,), (MS,)),
            lax.dynamic_slice(n_idx, (s * MS,), (MS,)))
        et3 = lax.dynamic_slice(et, (s * MS,), (MS,)).reshape(
            MS // _BM, _BM, 1)
        out = out + _tc_loss(heads, tails, negs, et3, wab, wbb, w1b, b1r,
                             br1r, w2t, b2s, wr2=Wr2, br2r=br2r, m_total=M)
    return out.reshape(())
